# baseline (speedup 1.0000x reference)
import functools

import jax
import jax.numpy as jnp
from jax import lax
from jax.experimental import pallas as pl
from jax.experimental.pallas import tpu as pltpu
from jax.experimental.pallas import tpu_sc as plsc

_B = 1024
_N = 100000
_L = 10
_LANES = 16

_info = plsc.get_sparse_core_info()
_NC = _info.num_cores
_NW = _NC * _info.num_subcores
_ROWS_PER_W = _B // _NW

_mesh = plsc.VectorSubcoreMesh(core_axis_name="c", subcore_axis_name="s")


@functools.partial(
    pl.kernel,
    out_type=jax.ShapeDtypeStruct((_B * _N,), jnp.float32),
    mesh=_mesh,
    scratch_types=[
        pltpu.VMEM((_ROWS_PER_W * _LANES,), jnp.int32),
        pltpu.VMEM((_N,), jnp.float32),
        pltpu.VMEM((_LANES,), jnp.float32),
        pltpu.SemaphoreType.DMA,
        pltpu.SemaphoreType.DMA,
    ],
    compiler_params=pltpu.CompilerParams(needs_layout_passes=False),
)
def _ten_hot(x_hbm, out_hbm, xv, rowbuf, ones_v, sem_z, sem_s):
    wid = lax.axis_index("s") * _NC + lax.axis_index("c")
    base = wid * _ROWS_PER_W

    pltpu.sync_copy(x_hbm.at[pl.ds(base * _LANES, _ROWS_PER_W * _LANES)], xv)

    zeros16 = jnp.zeros((_LANES,), jnp.float32)
    ones_v[...] = jnp.ones((_LANES,), jnp.float32)

    def zero_body(i, c):
        for k in range(10):
            rowbuf[pl.ds(i * 160 + k * _LANES, _LANES)] = zeros16
        return c

    lax.fori_loop(0, _N // 160, zero_body, 0)

    def fire(r, c):
        pltpu.async_copy(rowbuf, out_hbm.at[pl.ds((base + r) * _N, _N)], sem_z)
        return c

    lax.fori_loop(0, _ROWS_PER_W, fire, 0)

    def drain_zero(r, c):
        pltpu.make_async_copy(rowbuf, out_hbm.at[pl.ds(0, _N)], sem_z).wait()
        return c

    lax.fori_loop(0, _ROWS_PER_W, drain_zero, 0)

    def scatter(r, c):
        toks = xv[pl.ds(r * _LANES, _LANES)]
        idx = toks + (base + r) * _N
        pltpu.async_copy(ones_v, out_hbm.at[idx], sem_s)
        return c

    lax.fori_loop(0, _ROWS_PER_W, scatter, 0)

    def drain_scatter(r, c):
        pltpu.make_async_copy(ones_v, out_hbm.at[pl.ds(0, _LANES)], sem_s).wait()
        return c

    lax.fori_loop(0, _ROWS_PER_W, drain_scatter, 0)


def kernel(x):
    xp = jnp.concatenate(
        [x, jnp.broadcast_to(x[:, :1], (_B, _LANES - _L))], axis=1
    )
    return _ten_hot(xp.reshape(-1)).reshape(_B, _N)


# speedup vs baseline: 2.0262x; 2.0262x over previous
import functools

import jax
import jax.numpy as jnp
from jax import lax
from jax.experimental import pallas as pl
from jax.experimental.pallas import tpu as pltpu
from jax.experimental.pallas import tpu_sc as plsc

_B = 1024
_N = 100000
_L = 10
_LANES = 16
_SEG = 128
_G = 4
_RPG = 8

_info = plsc.get_sparse_core_info()
_NC = _info.num_cores
_NW = _NC * _info.num_subcores
_ROWS_PER_W = _B // _NW

_mesh = plsc.VectorSubcoreMesh(core_axis_name="c", subcore_axis_name="s")


@functools.partial(
    pl.kernel,
    out_type=jax.ShapeDtypeStruct((_B, _N), jnp.float32),
    mesh=_mesh,
    scratch_types=[
        pltpu.VMEM((_ROWS_PER_W * _LANES,), jnp.int32),
        pltpu.VMEM((_N,), jnp.float32),
        pltpu.VMEM((_RPG * _LANES, _SEG), jnp.float32),
        pltpu.SemaphoreType.DMA,
        pltpu.SemaphoreType.DMA,
        pltpu.SemaphoreType.DMA,
        pltpu.SemaphoreType.DMA,
        pltpu.SemaphoreType.DMA,
    ],
    compiler_params=pltpu.CompilerParams(needs_layout_passes=False),
)
def _ten_hot(x_hbm, out_hbm, xv, rowbuf, patch, s0, s1, s2, s3, sp):
    zsem = (s0, s1, s2, s3)
    wid = lax.axis_index("s") * _NC + lax.axis_index("c")
    base = wid * _ROWS_PER_W

    pltpu.sync_copy(x_hbm.at[pl.ds(base * _LANES, _ROWS_PER_W * _LANES)], xv)

    zeros16 = jnp.zeros((_LANES,), jnp.float32)
    ones16 = jnp.ones((_LANES,), jnp.float32)
    iota16 = lax.iota(jnp.int32, _LANES)

    def zero_body(i, c):
        for k in range(10):
            rowbuf[pl.ds(i * 160 + k * _LANES, _LANES)] = zeros16
        return c

    lax.fori_loop(0, _N // 160, zero_body, 0)

    def zero_patch(i, c):
        for k in range(_SEG // _LANES):
            patch[i, pl.ds(k * _LANES, _LANES)] = zeros16
        return c

    lax.fori_loop(0, _RPG * _LANES, zero_patch, 0)

    for g in range(_G):

        def fire(r, c, g=g):
            pltpu.async_copy(rowbuf, out_hbm.at[base + g * _RPG + r], zsem[g])
            return c

        lax.fori_loop(0, _RPG, fire, 0)

    def _scalar(vec, j):
        return jnp.max(jnp.where(iota16 == j, vec, -1))

    for g in range(_G):
        def drain_zero(r, c, g=g):
            pltpu.make_async_copy(rowbuf, out_hbm.at[0], zsem[g]).wait()
            return c

        lax.fori_loop(0, _RPG, drain_zero, 0)

        if g >= 1:
            def drain_patch(i, c):
                pltpu.make_async_copy(
                    patch.at[0], out_hbm.at[0].at[pl.ds(0, _SEG)], sp
                ).wait()
                return c

            lax.fori_loop(0, _RPG * _LANES, drain_patch, 0)

        def row_fn(ri, c, g=g):
            r = g * _RPG + ri
            toks = xv[pl.ds(r * _LANES, _LANES)]
            seg = lax.shift_right_logical(toks, 7)
            srow = ri * _LANES

            if g >= 1:
                toks_prev = xv[pl.ds((r - _RPG) * _LANES, _LANES)]
                for j in range(_LANES):
                    tp = _scalar(toks_prev, j)
                    offp = jnp.bitwise_and(tp, _SEG - 1)
                    plsc.store_scatter(
                        patch,
                        [srow + iota16, jnp.broadcast_to(offp, (_LANES,))],
                        zeros16,
                    )

            toks_s = []
            for j in range(_LANES):
                tj = _scalar(toks, j)
                toks_s.append(tj)
                offj = jnp.bitwise_and(tj, _SEG - 1)
                m = seg == lax.shift_right_logical(tj, 7)
                plsc.store_scatter(
                    patch,
                    [srow + iota16, jnp.broadcast_to(offj, (_LANES,))],
                    ones16,
                    mask=m,
                )
            for j in range(_LANES):
                segj = lax.shift_right_logical(toks_s[j], 7)
                start = pl.multiple_of(segj * _SEG, _SEG)
                pltpu.async_copy(
                    patch.at[srow + j],
                    out_hbm.at[base + r].at[pl.ds(start, _SEG)],
                    sp,
                )
            return c

        lax.fori_loop(0, _RPG, row_fn, 0)

    def drain_patch_final(i, c):
        pltpu.make_async_copy(
            patch.at[0], out_hbm.at[0].at[pl.ds(0, _SEG)], sp
        ).wait()
        return c

    lax.fori_loop(0, _RPG * _LANES, drain_patch_final, 0)


def kernel(x):
    xp = jnp.concatenate(
        [x, jnp.broadcast_to(x[:, :1], (_B, _LANES - _L))], axis=1
    )
    return _ten_hot(xp.reshape(-1))


# speedup vs baseline: 2.3948x; 1.1819x over previous
import functools

import jax
import jax.numpy as jnp
from jax import lax
from jax.experimental import pallas as pl
from jax.experimental.pallas import tpu as pltpu
from jax.experimental.pallas import tpu_sc as plsc

_B = 1024
_N = 100000
_QTR = _N // 4
_WPR = _B // 32
_L = 10
_LANES = 16
_CROWS = 16
_CWORDS = _CROWS * _WPR
_ZB = 4096

_info = plsc.get_sparse_core_info()
_NC = _info.num_cores
_NS = _info.num_subcores
_BPW = _B // _NS

_mesh = plsc.VectorSubcoreMesh(core_axis_name="c", subcore_axis_name="s")


@functools.partial(
    pl.kernel,
    out_type=jax.ShapeDtypeStruct((_N, _B), jnp.float32),
    mesh=_mesh,
    scratch_types=[
        pltpu.VMEM((_BPW * _LANES,), jnp.int32),
        pltpu.VMEM((_BPW, _LANES), jnp.int32),
        pltpu.VMEM((_BPW, _LANES), jnp.int32),
        pltpu.VMEM((_CROWS, _B), jnp.float32),
        pltpu.VMEM((_CROWS, _B), jnp.float32),
        pltpu.VMEM((_CWORDS,), jnp.int32),
        pltpu.VMEM((_CWORDS,), jnp.int32),
        pltpu.VMEM((_ZB,), jnp.int32),
        pltpu.VMEM_SHARED((_QTR * _WPR,), jnp.int32),
        pltpu.SemaphoreType.DMA,
        pltpu.SemaphoreType.DMA,
        pltpu.SemaphoreType.DMA,
        pltpu.SemaphoreType.DMA,
    ],
    compiler_params=pltpu.CompilerParams(needs_layout_passes=False),
)
def _ten_hot(
    x_hbm,
    out_hbm,
    xv,
    idxbuf,
    valbuf,
    ping,
    pong,
    bs_a,
    bs_b,
    zbuf,
    bitmap,
    sem_a,
    sem_b,
    sem_s,
    sem_i,
):
    c = lax.axis_index("c")
    s = lax.axis_index("s")

    pltpu.sync_copy(x_hbm.at[pl.ds(s * _BPW * _LANES, _BPW * _LANES)], xv)

    zeros16f = jnp.zeros((_LANES,), jnp.float32)
    zeros16i = jnp.zeros((_LANES,), jnp.int32)
    iota16 = lax.iota(jnp.int32, _LANES)

    def _extract(vec, j):
        return jnp.sum(jnp.where(iota16 == j, vec, 0))

    def zrow(r, carry):
        for k in range(_B // _LANES):
            ping[r, pl.ds(k * _LANES, _LANES)] = zeros16f
            pong[r, pl.ds(k * _LANES, _LANES)] = zeros16f
        return carry

    lax.fori_loop(0, _CROWS, zrow, 0)

    def zzb(i, carry):
        for k in range(8):
            zbuf[pl.ds(i * 128 + k * _LANES, _LANES)] = zeros16i
        return carry

    lax.fori_loop(0, _ZB // 128, zzb, 0)

    def expand(bs, buf, nwords, set_ones):
        def vbody(v, carry):
            w16 = bs[pl.ds(v * _LANES, _LANES)]
            nz = jnp.sum((w16 != 0).astype(jnp.int32))

            @pl.when(nz > 0)
            def _():
                def lbody(lane, lcarry):
                    wl = _extract(w16, lane)

                    @pl.when(wl != 0)
                    def _():
                        f = v * _LANES + lane
                        row = f // _WPR
                        colb = (f % _WPR) * 32
                        if set_ones:
                            wv = jnp.broadcast_to(wl, (_LANES,))
                            lo = (
                                lax.shift_right_logical(wv, iota16) & 1
                            ).astype(jnp.float32)
                            hi = (
                                lax.shift_right_logical(wv, iota16 + 16) & 1
                            ).astype(jnp.float32)
                            buf[row, pl.ds(colb, _LANES)] = lo
                            buf[row, pl.ds(colb + _LANES, _LANES)] = hi
                        else:
                            buf[row, pl.ds(colb, _LANES)] = zeros16f
                            buf[row, pl.ds(colb + _LANES, _LANES)] = zeros16f

                    return lcarry

                lax.fori_loop(0, _LANES, lbody, 0)

            return carry

        lax.fori_loop(0, nwords // _LANES, vbody, 0)

    per_sub = _QTR * _WPR // _NS
    nfull = per_sub // _ZB
    rem = per_sub - nfull * _ZB

    for q in range(2):
        qbase = c * (2 * _QTR) + q * _QTR

        def zb(i, carry):
            pltpu.async_copy(
                zbuf, bitmap.at[pl.ds(s * per_sub + i * _ZB, _ZB)], sem_i
            )
            return carry

        lax.fori_loop(0, nfull, zb, 0)
        pltpu.async_copy(
            zbuf.at[pl.ds(0, rem)],
            bitmap.at[pl.ds(s * per_sub + nfull * _ZB, rem)],
            sem_i,
        )

        def zb_drain(i, carry):
            pltpu.make_async_copy(
                zbuf, bitmap.at[pl.ds(0, _ZB)], sem_i
            ).wait()
            return carry

        lax.fori_loop(0, nfull, zb_drain, 0)
        pltpu.make_async_copy(
            zbuf.at[pl.ds(0, rem)], bitmap.at[pl.ds(0, rem)], sem_i
        ).wait()

        plsc.subcore_barrier()

        def prep(bi, carry):
            toks = xv[pl.ds(bi * _LANES, _LANES)]
            t_loc = toks - qbase
            inq = (t_loc >= 0) & (t_loc < _QTR)
            t_cl = jnp.clip(t_loc, 0, _QTR - 1)
            dup = jnp.zeros((_LANES,), jnp.bool_)
            for j in range(_LANES - 1):
                tj = _extract(toks, j)
                dup = dup | ((toks == tj) & (iota16 > j))
            b = s * _BPW + bi
            bit = jnp.int32(1) << (b % 32)
            vals = jnp.where(inq & jnp.logical_not(dup), bit, 0)
            idxbuf[bi] = t_cl * _WPR + b // 32
            valbuf[bi] = vals
            return carry

        lax.fori_loop(0, _BPW, prep, 0)

        def fire_sc(bi, carry):
            pltpu.async_copy(
                valbuf.at[bi], bitmap.at[idxbuf.at[bi]], sem_s, add=True
            )
            return carry

        lax.fori_loop(0, _BPW, fire_sc, 0)

        def drain_sc(bi, carry):
            pltpu.make_async_copy(
                valbuf.at[0], bitmap.at[idxbuf.at[0]], sem_s
            ).wait()
            return carry

        lax.fori_loop(0, _BPW, drain_sc, 0)

        plsc.subcore_barrier()

        start_blk = s * 195 + jnp.minimum(s, 5)
        r0 = start_blk * 8
        nch = 98 - (s >= 5).astype(jnp.int32)
        has_tail = s >= 5

        def step(ci, buf, bs, sem):
            @pl.when(ci >= 2)
            def _():
                pltpu.make_async_copy(
                    buf, out_hbm.at[pl.ds(0, _CROWS)], sem
                ).wait()
                expand(bs, buf, _CWORDS, set_ones=False)

            pltpu.sync_copy(
                bitmap.at[pl.ds(r0 * _WPR + ci * _CWORDS, _CWORDS)], bs
            )
            expand(bs, buf, _CWORDS, set_ones=True)
            pltpu.async_copy(
                buf,
                out_hbm.at[pl.ds(qbase + r0 + ci * _CROWS, _CROWS)],
                sem,
            )

        def chunk(ci, carry):
            @pl.when(ci % 2 == 0)
            def _():
                step(ci, ping, bs_a, sem_a)

            @pl.when(ci % 2 == 1)
            def _():
                step(ci, pong, bs_b, sem_b)

            return carry

        lax.fori_loop(0, nch, chunk, 0)

        pltpu.make_async_copy(
            ping, out_hbm.at[pl.ds(0, _CROWS)], sem_a
        ).wait()
        pltpu.make_async_copy(
            pong, out_hbm.at[pl.ds(0, _CROWS)], sem_b
        ).wait()

        expand(bs_a, ping, _CWORDS, set_ones=False)
        expand(bs_b, pong, _CWORDS, set_ones=False)

        @pl.when(has_tail)
        def _():
            tr = r0 + nch * _CROWS
            pltpu.sync_copy(
                bitmap.at[pl.ds(tr * _WPR, 8 * _WPR)],
                bs_a.at[pl.ds(0, 8 * _WPR)],
            )
            expand(bs_a, ping, 8 * _WPR, set_ones=True)
            pltpu.sync_copy(
                ping.at[pl.ds(0, 8)], out_hbm.at[pl.ds(qbase + tr, 8)]
            )
            expand(bs_a, ping, 8 * _WPR, set_ones=False)

        plsc.subcore_barrier()


def kernel(x):
    xp = jnp.concatenate(
        [x, jnp.broadcast_to(x[:, :1], (_B, _LANES - _L))], axis=1
    )
    return _ten_hot(xp.reshape(-1)).T


# speedup vs baseline: 2.5039x; 1.0456x over previous
import functools

import jax
import jax.numpy as jnp
from jax import lax
from jax.experimental import pallas as pl
from jax.experimental.pallas import tpu as pltpu
from jax.experimental.pallas import tpu_sc as plsc

_B = 1024
_N = 100000
_QTR = _N // 4
_WPR = _B // 32
_L = 10
_LANES = 16
_CROWS = 16
_CWORDS = _CROWS * _WPR
_ZB = 4096

_info = plsc.get_sparse_core_info()
_NC = _info.num_cores
_NS = _info.num_subcores
_BPW = _B // _NS

_mesh = plsc.VectorSubcoreMesh(core_axis_name="c", subcore_axis_name="s")


@functools.partial(
    pl.kernel,
    out_type=jax.ShapeDtypeStruct((_N, _B), jnp.float32),
    mesh=_mesh,
    scratch_types=[
        pltpu.VMEM((_BPW * _LANES,), jnp.int32),
        pltpu.VMEM((_BPW, _LANES), jnp.int32),
        pltpu.VMEM((_BPW, _LANES), jnp.int32),
        pltpu.VMEM((_CROWS, _B), jnp.float32),
        pltpu.VMEM((_CROWS, _B), jnp.float32),
        pltpu.VMEM((_CWORDS,), jnp.int32),
        pltpu.VMEM((_CWORDS,), jnp.int32),
        pltpu.VMEM((_CWORDS,), jnp.int32),
        pltpu.VMEM((_CWORDS,), jnp.int32),
        pltpu.VMEM((_ZB,), jnp.int32),
        pltpu.VMEM_SHARED((_QTR * _WPR,), jnp.int32),
        pltpu.SemaphoreType.DMA,
        pltpu.SemaphoreType.DMA,
        pltpu.SemaphoreType.DMA,
        pltpu.SemaphoreType.DMA,
        pltpu.SemaphoreType.DMA,
        pltpu.SemaphoreType.DMA,
        pltpu.SemaphoreType.DMA,
        pltpu.SemaphoreType.DMA,
    ],
    compiler_params=pltpu.CompilerParams(needs_layout_passes=False),
)
def _ten_hot(
    x_hbm,
    out_hbm,
    xv,
    idxbuf,
    valbuf,
    ping,
    pong,
    bs_a0,
    bs_a1,
    bs_b0,
    bs_b1,
    zbuf,
    bitmap,
    sem_a,
    sem_b,
    sem_s,
    sem_i,
    sem_sa0,
    sem_sa1,
    sem_sb0,
    sem_sb1,
):
    c = lax.axis_index("c")
    s = lax.axis_index("s")

    pltpu.sync_copy(x_hbm.at[pl.ds(s * _BPW * _LANES, _BPW * _LANES)], xv)

    zeros16f = jnp.zeros((_LANES,), jnp.float32)
    zeros16i = jnp.zeros((_LANES,), jnp.int32)
    iota16 = lax.iota(jnp.int32, _LANES)

    def _extract(vec, j):
        return jnp.sum(jnp.where(iota16 == j, vec, 0))

    def zrow(r, carry):
        for k in range(_B // _LANES):
            ping[r, pl.ds(k * _LANES, _LANES)] = zeros16f
            pong[r, pl.ds(k * _LANES, _LANES)] = zeros16f
        return carry

    lax.fori_loop(0, _CROWS, zrow, 0)

    def zzb(i, carry):
        for k in range(8):
            zbuf[pl.ds(i * 128 + k * _LANES, _LANES)] = zeros16i
        return carry

    lax.fori_loop(0, _ZB // 128, zzb, 0)

    def expand(bs, buf, nwords, set_ones):
        def vbody(v, carry):
            w16 = bs[pl.ds(v * _LANES, _LANES)]
            nz = jnp.sum((w16 != 0).astype(jnp.int32))

            @pl.when(nz > 0)
            def _():
                def lbody(lane, lcarry):
                    wl = _extract(w16, lane)

                    @pl.when(wl != 0)
                    def _():
                        f = v * _LANES + lane
                        row = f // _WPR
                        colb = (f % _WPR) * 32
                        if set_ones:
                            wv = jnp.broadcast_to(wl, (_LANES,))
                            lo = (
                                lax.shift_right_logical(wv, iota16) & 1
                            ).astype(jnp.float32)
                            hi = (
                                lax.shift_right_logical(wv, iota16 + 16) & 1
                            ).astype(jnp.float32)
                            buf[row, pl.ds(colb, _LANES)] = lo
                            buf[row, pl.ds(colb + _LANES, _LANES)] = hi
                        else:
                            buf[row, pl.ds(colb, _LANES)] = zeros16f
                            buf[row, pl.ds(colb + _LANES, _LANES)] = zeros16f

                    return lcarry

                lax.fori_loop(0, _LANES, lbody, 0)

            return carry

        lax.fori_loop(0, nwords // _LANES, vbody, 0)

    per_sub = _QTR * _WPR // _NS
    nfull = per_sub // _ZB
    rem = per_sub - nfull * _ZB

    for q in range(2):
        qbase = c * (2 * _QTR) + q * _QTR

        def zb(i, carry):
            pltpu.async_copy(
                zbuf, bitmap.at[pl.ds(s * per_sub + i * _ZB, _ZB)], sem_i
            )
            return carry

        lax.fori_loop(0, nfull, zb, 0)
        pltpu.async_copy(
            zbuf.at[pl.ds(0, rem)],
            bitmap.at[pl.ds(s * per_sub + nfull * _ZB, rem)],
            sem_i,
        )

        def zb_drain(i, carry):
            pltpu.make_async_copy(
                zbuf, bitmap.at[pl.ds(0, _ZB)], sem_i
            ).wait()
            return carry

        lax.fori_loop(0, nfull, zb_drain, 0)
        pltpu.make_async_copy(
            zbuf.at[pl.ds(0, rem)], bitmap.at[pl.ds(0, rem)], sem_i
        ).wait()

        plsc.subcore_barrier()

        def prep(bi, carry):
            toks = xv[pl.ds(bi * _LANES, _LANES)]
            t_loc = toks - qbase
            inq = (t_loc >= 0) & (t_loc < _QTR)
            t_cl = jnp.clip(t_loc, 0, _QTR - 1)
            dup = jnp.zeros((_LANES,), jnp.bool_)
            for j in range(_LANES - 1):
                tj = _extract(toks, j)
                dup = dup | ((toks == tj) & (iota16 > j))
            b = s * _BPW + bi
            bit = jnp.int32(1) << (b % 32)
            vals = jnp.where(inq & jnp.logical_not(dup), bit, 0)
            idxbuf[bi] = t_cl * _WPR + b // 32
            valbuf[bi] = vals
            return carry

        lax.fori_loop(0, _BPW, prep, 0)

        def fire_sc(bi, carry):
            pltpu.async_copy(
                valbuf.at[bi], bitmap.at[idxbuf.at[bi]], sem_s, add=True
            )
            return carry

        lax.fori_loop(0, _BPW, fire_sc, 0)

        def drain_sc(bi, carry):
            pltpu.make_async_copy(
                valbuf.at[0], bitmap.at[idxbuf.at[0]], sem_s
            ).wait()
            return carry

        lax.fori_loop(0, _BPW, drain_sc, 0)

        plsc.subcore_barrier()

        start_blk = s * 195 + jnp.minimum(s, 5)
        r0 = start_blk * 8
        nch = 98 - (s >= 5).astype(jnp.int32)
        has_tail = s >= 5

        def slice_at(ci):
            return bitmap.at[pl.ds(r0 * _WPR + ci * _CWORDS, _CWORDS)]

        def stage(ci, slot, sem):
            pltpu.async_copy(slice_at(ci), slot, sem)

        def stage_wait(slot, sem):
            pltpu.make_async_copy(
                bitmap.at[pl.ds(0, _CWORDS)], slot, sem
            ).wait()

        stage(0, bs_a0, sem_sa0)
        stage(1, bs_b0, sem_sb0)
        stage(2, bs_a1, sem_sa1)
        stage(3, bs_b1, sem_sb1)

        def step(ci, buf, sem_buf, s_set, sem_set, s_oth, sem_oth, refill):
            @pl.when(ci >= 2)
            def _():
                pltpu.make_async_copy(
                    buf, out_hbm.at[pl.ds(0, _CROWS)], sem_buf
                ).wait()
                expand(s_oth, buf, _CWORDS, set_ones=False)
                if refill:

                    @pl.when(ci + 2 < nch)
                    def _():
                        stage(ci + 2, s_oth, sem_oth)

            stage_wait(s_set, sem_set)
            expand(s_set, buf, _CWORDS, set_ones=True)
            pltpu.async_copy(
                buf,
                out_hbm.at[pl.ds(qbase + r0 + ci * _CROWS, _CROWS)],
                sem_buf,
            )

        def quad(qi, carry):
            ci = 4 * qi
            step(ci, ping, sem_a, bs_a0, sem_sa0, bs_a1, sem_sa1, True)
            step(ci + 1, pong, sem_b, bs_b0, sem_sb0, bs_b1, sem_sb1, True)
            step(ci + 2, ping, sem_a, bs_a1, sem_sa1, bs_a0, sem_sa0, True)
            step(ci + 3, pong, sem_b, bs_b1, sem_sb1, bs_b0, sem_sb0, True)
            return carry

        lax.fori_loop(0, 24, quad, 0)

        step(96, ping, sem_a, bs_a0, sem_sa0, bs_a1, sem_sa1, False)

        @pl.when(nch > 97)
        def _():
            step(97, pong, sem_b, bs_b0, sem_sb0, bs_b1, sem_sb1, False)

        pltpu.make_async_copy(
            ping, out_hbm.at[pl.ds(0, _CROWS)], sem_a
        ).wait()
        pltpu.make_async_copy(
            pong, out_hbm.at[pl.ds(0, _CROWS)], sem_b
        ).wait()

        expand(bs_a0, ping, _CWORDS, set_ones=False)

        @pl.when(nch > 97)
        def _():
            expand(bs_b0, pong, _CWORDS, set_ones=False)

        @pl.when(nch <= 97)
        def _():
            expand(bs_b1, pong, _CWORDS, set_ones=False)

        @pl.when(has_tail)
        def _():
            tr = r0 + nch * _CROWS
            pltpu.sync_copy(
                bitmap.at[pl.ds(tr * _WPR, 8 * _WPR)],
                bs_a0.at[pl.ds(0, 8 * _WPR)],
            )
            expand(bs_a0, ping, 8 * _WPR, set_ones=True)
            pltpu.sync_copy(
                ping.at[pl.ds(0, 8)], out_hbm.at[pl.ds(qbase + tr, 8)]
            )
            expand(bs_a0, ping, 8 * _WPR, set_ones=False)

        plsc.subcore_barrier()


def kernel(x):
    xp = jnp.concatenate(
        [x, jnp.broadcast_to(x[:, :1], (_B, _LANES - _L))], axis=1
    )
    return _ten_hot(xp.reshape(-1)).T


# speedup vs baseline: 3.6671x; 1.4645x over previous
import functools

import jax
import jax.numpy as jnp
from jax import lax
from jax.experimental import pallas as pl
from jax.experimental.pallas import tpu as pltpu
from jax.experimental.pallas import tpu_sc as plsc

_B = 1024
_N = 100000
_QTR = _N // 4
_WPR = _B // 32
_L = 10
_LANES = 16
_CROWS = 16
_CWORDS = _CROWS * _WPR
_ZB = 4096

_info = plsc.get_sparse_core_info()
_NC = _info.num_cores
_NS = _info.num_subcores
_BPW = _B // _NS

_mesh = plsc.VectorSubcoreMesh(core_axis_name="c", subcore_axis_name="s")


@functools.partial(
    pl.kernel,
    out_type=jax.ShapeDtypeStruct((_N, _B), jnp.float32),
    mesh=_mesh,
    scratch_types=[
        pltpu.VMEM((_BPW * _LANES,), jnp.int32),
        pltpu.VMEM((_BPW, _LANES), jnp.int32),
        pltpu.VMEM((_BPW, _LANES), jnp.int32),
        pltpu.VMEM((_CROWS, _B), jnp.float32),
        pltpu.VMEM((_CROWS, _B), jnp.float32),
        pltpu.VMEM((_CWORDS,), jnp.int32),
        pltpu.VMEM((_CWORDS,), jnp.int32),
        pltpu.VMEM((_CWORDS,), jnp.int32),
        pltpu.VMEM((_CWORDS,), jnp.int32),
        pltpu.VMEM((_ZB,), jnp.int32),
        pltpu.VMEM_SHARED((_QTR * _WPR,), jnp.int32),
        pltpu.SemaphoreType.DMA,
        pltpu.SemaphoreType.DMA,
        pltpu.SemaphoreType.DMA,
        pltpu.SemaphoreType.DMA,
        pltpu.SemaphoreType.DMA,
        pltpu.SemaphoreType.DMA,
        pltpu.SemaphoreType.DMA,
        pltpu.SemaphoreType.DMA,
    ],
    compiler_params=pltpu.CompilerParams(needs_layout_passes=False),
)
def _ten_hot(
    x_hbm,
    out_hbm,
    xv,
    idxbuf,
    valbuf,
    ping,
    pong,
    bs_a0,
    bs_a1,
    bs_b0,
    bs_b1,
    zbuf,
    bitmap,
    sem_a,
    sem_b,
    sem_s,
    sem_i,
    sem_sa0,
    sem_sa1,
    sem_sb0,
    sem_sb1,
):
    c = lax.axis_index("c")
    s = lax.axis_index("s")

    pltpu.sync_copy(x_hbm.at[pl.ds(s * _BPW * _LANES, _BPW * _LANES)], xv)

    zeros16f = jnp.zeros((_LANES,), jnp.float32)
    zeros16i = jnp.zeros((_LANES,), jnp.int32)
    iota16 = lax.iota(jnp.int32, _LANES)

    def _extract(vec, j):
        return jnp.sum(jnp.where(iota16 == j, vec, 0))

    def zrow(r, carry):
        for k in range(_B // _LANES):
            ping[r, pl.ds(k * _LANES, _LANES)] = zeros16f
            pong[r, pl.ds(k * _LANES, _LANES)] = zeros16f
        return carry

    lax.fori_loop(0, _CROWS, zrow, 0)

    def zzb(i, carry):
        for k in range(8):
            zbuf[pl.ds(i * 128 + k * _LANES, _LANES)] = zeros16i
        return carry

    lax.fori_loop(0, _ZB // 128, zzb, 0)

    def expand(bs, buf, nwords, set_ones):
        def vbody(v, carry):
            w16 = bs[pl.ds(v * _LANES, _LANES)]
            nz = jnp.sum((w16 != 0).astype(jnp.int32))

            @pl.when(nz > 0)
            def _():
                def lbody(lane, lcarry):
                    wl = _extract(w16, lane)

                    @pl.when(wl != 0)
                    def _():
                        f = v * _LANES + lane
                        row = f // _WPR
                        colb = (f % _WPR) * 32
                        if set_ones:
                            wv = jnp.broadcast_to(wl, (_LANES,))
                            lo = (
                                lax.shift_right_logical(wv, iota16) & 1
                            ).astype(jnp.float32)
                            hi = (
                                lax.shift_right_logical(wv, iota16 + 16) & 1
                            ).astype(jnp.float32)
                            buf[row, pl.ds(colb, _LANES)] = lo
                            buf[row, pl.ds(colb + _LANES, _LANES)] = hi
                        else:
                            buf[row, pl.ds(colb, _LANES)] = zeros16f
                            buf[row, pl.ds(colb + _LANES, _LANES)] = zeros16f

                    return lcarry

                lax.fori_loop(0, _LANES, lbody, 0)

            return carry

        lax.fori_loop(0, nwords // _LANES, vbody, 0)

    def expand_fast(bs, buf, set_ones):
        def gbody(g, carry):
            acc = bs[pl.ds(g * 128, _LANES)]
            for k in range(1, 8):
                acc = acc | bs[pl.ds(g * 128 + k * _LANES, _LANES)]
            nzg = jnp.sum((acc != 0).astype(jnp.int32))

            @pl.when(nzg > 0)
            def _():
                def vbody(vv, vcarry):
                    v = g * 8 + vv
                    w16 = bs[pl.ds(v * _LANES, _LANES)]
                    nz = jnp.sum((w16 != 0).astype(jnp.int32))

                    @pl.when(nz > 0)
                    def _():
                        def lbody(lane, lcarry):
                            wl = _extract(w16, lane)

                            @pl.when(wl != 0)
                            def _():
                                f = v * _LANES + lane
                                row = f // _WPR
                                colb = (f % _WPR) * 32
                                if set_ones:
                                    wv = jnp.broadcast_to(wl, (_LANES,))
                                    lo = (
                                        lax.shift_right_logical(wv, iota16)
                                        & 1
                                    ).astype(jnp.float32)
                                    hi = (
                                        lax.shift_right_logical(
                                            wv, iota16 + 16
                                        )
                                        & 1
                                    ).astype(jnp.float32)
                                    buf[row, pl.ds(colb, _LANES)] = lo
                                    buf[row, pl.ds(colb + _LANES, _LANES)] = (
                                        hi
                                    )
                                else:
                                    buf[row, pl.ds(colb, _LANES)] = zeros16f
                                    buf[row, pl.ds(colb + _LANES, _LANES)] = (
                                        zeros16f
                                    )

                            return lcarry

                        lax.fori_loop(0, _LANES, lbody, 0)

                    return vcarry

                lax.fori_loop(0, 8, vbody, 0)

            return carry

        lax.fori_loop(0, _CWORDS // 128, gbody, 0)

    per_sub = _QTR * _WPR // _NS
    nfull = per_sub // _ZB
    rem = per_sub - nfull * _ZB

    for q in range(2):
        qbase = c * (2 * _QTR) + q * _QTR

        def zb(i, carry):
            pltpu.async_copy(
                zbuf, bitmap.at[pl.ds(s * per_sub + i * _ZB, _ZB)], sem_i
            )
            return carry

        lax.fori_loop(0, nfull, zb, 0)
        pltpu.async_copy(
            zbuf.at[pl.ds(0, rem)],
            bitmap.at[pl.ds(s * per_sub + nfull * _ZB, rem)],
            sem_i,
        )

        def zb_drain(i, carry):
            pltpu.make_async_copy(
                zbuf, bitmap.at[pl.ds(0, _ZB)], sem_i
            ).wait()
            return carry

        lax.fori_loop(0, nfull, zb_drain, 0)
        pltpu.make_async_copy(
            zbuf.at[pl.ds(0, rem)], bitmap.at[pl.ds(0, rem)], sem_i
        ).wait()

        plsc.subcore_barrier()

        def prep(bi, carry):
            toks = xv[pl.ds(bi * _LANES, _LANES)]
            t_loc = toks - qbase
            inq = (t_loc >= 0) & (t_loc < _QTR)
            t_cl = jnp.clip(t_loc, 0, _QTR - 1)
            dup = jnp.zeros((_LANES,), jnp.bool_)
            for j in range(_LANES - 1):
                tj = _extract(toks, j)
                dup = dup | ((toks == tj) & (iota16 > j))
            b = s * _BPW + bi
            bit = jnp.int32(1) << (b % 32)
            vals = jnp.where(inq & jnp.logical_not(dup), bit, 0)
            idxbuf[bi] = t_cl * _WPR + b // 32
            valbuf[bi] = vals
            return carry

        lax.fori_loop(0, _BPW, prep, 0)

        def fire_sc(bi, carry):
            pltpu.async_copy(
                valbuf.at[bi], bitmap.at[idxbuf.at[bi]], sem_s, add=True
            )
            return carry

        lax.fori_loop(0, _BPW, fire_sc, 0)

        def drain_sc(bi, carry):
            pltpu.make_async_copy(
                valbuf.at[0], bitmap.at[idxbuf.at[0]], sem_s
            ).wait()
            return carry

        lax.fori_loop(0, _BPW, drain_sc, 0)

        plsc.subcore_barrier()

        start_blk = s * 195 + jnp.minimum(s, 5)
        r0 = start_blk * 8
        nch = 98 - (s >= 5).astype(jnp.int32)
        has_tail = s >= 5

        def slice_at(ci):
            return bitmap.at[pl.ds(r0 * _WPR + ci * _CWORDS, _CWORDS)]

        def stage(ci, slot, sem):
            pltpu.async_copy(slice_at(ci), slot, sem)

        def stage_wait(slot, sem):
            pltpu.make_async_copy(
                bitmap.at[pl.ds(0, _CWORDS)], slot, sem
            ).wait()

        stage(0, bs_a0, sem_sa0)
        stage(1, bs_b0, sem_sb0)
        stage(2, bs_a1, sem_sa1)
        stage(3, bs_b1, sem_sb1)

        def step(ci, buf, sem_buf, s_set, sem_set, s_oth, sem_oth, refill):
            @pl.when(ci >= 2)
            def _():
                pltpu.make_async_copy(
                    buf, out_hbm.at[pl.ds(0, _CROWS)], sem_buf
                ).wait()
                expand_fast(s_oth, buf, set_ones=False)
                if refill:

                    @pl.when(ci + 2 < nch)
                    def _():
                        stage(ci + 2, s_oth, sem_oth)

            stage_wait(s_set, sem_set)
            expand_fast(s_set, buf, set_ones=True)
            pltpu.async_copy(
                buf,
                out_hbm.at[pl.ds(qbase + r0 + ci * _CROWS, _CROWS)],
                sem_buf,
            )

        def quad(qi, carry):
            ci = 4 * qi
            step(ci, ping, sem_a, bs_a0, sem_sa0, bs_a1, sem_sa1, True)
            step(ci + 1, pong, sem_b, bs_b0, sem_sb0, bs_b1, sem_sb1, True)
            step(ci + 2, ping, sem_a, bs_a1, sem_sa1, bs_a0, sem_sa0, True)
            step(ci + 3, pong, sem_b, bs_b1, sem_sb1, bs_b0, sem_sb0, True)
            return carry

        lax.fori_loop(0, 24, quad, 0)

        step(96, ping, sem_a, bs_a0, sem_sa0, bs_a1, sem_sa1, False)

        @pl.when(nch > 97)
        def _():
            step(97, pong, sem_b, bs_b0, sem_sb0, bs_b1, sem_sb1, False)

        pltpu.make_async_copy(
            ping, out_hbm.at[pl.ds(0, _CROWS)], sem_a
        ).wait()
        pltpu.make_async_copy(
            pong, out_hbm.at[pl.ds(0, _CROWS)], sem_b
        ).wait()

        expand(bs_a0, ping, _CWORDS, set_ones=False)

        @pl.when(nch > 97)
        def _():
            expand(bs_b0, pong, _CWORDS, set_ones=False)

        @pl.when(nch <= 97)
        def _():
            expand(bs_b1, pong, _CWORDS, set_ones=False)

        @pl.when(has_tail)
        def _():
            tr = r0 + nch * _CROWS
            pltpu.sync_copy(
                bitmap.at[pl.ds(tr * _WPR, 8 * _WPR)],
                bs_a0.at[pl.ds(0, 8 * _WPR)],
            )
            expand(bs_a0, ping, 8 * _WPR, set_ones=True)
            pltpu.sync_copy(
                ping.at[pl.ds(0, 8)], out_hbm.at[pl.ds(qbase + tr, 8)]
            )
            expand(bs_a0, ping, 8 * _WPR, set_ones=False)

        plsc.subcore_barrier()


def kernel(x):
    xp = jnp.concatenate(
        [x, jnp.broadcast_to(x[:, :1], (_B, _LANES - _L))], axis=1
    )
    return _ten_hot(xp.reshape(-1)).T
